# baseline (speedup 1.0000x reference)
import jax
import jax.numpy as jnp
import numpy as np
from jax.experimental import pallas as pl
from jax.experimental.pallas import tpu as pltpu

_N_BINS = 15
_B = 256
_C = 1000


def _ece_body(x_ref, t_ref, out_ref, acc_ref, *, n_total):
    step = pl.program_id(0)

    @pl.when(step == 0)
    def _init():
        acc_ref[...] = jnp.zeros_like(acc_ref)

    b = x_ref.shape[0]
    conf = jnp.ones((b, 1), dtype=jnp.float32)
    acc_row = jnp.zeros((b, 1), dtype=jnp.float32)
    t = t_ref[...]
    for j in range(3):
        x = x_ref[:, j, :]
        m = jnp.max(x, axis=1, keepdims=True)
        s = jnp.sum(jnp.exp(x - m), axis=1, keepdims=True)
        conf = conf * (1.0 / s)
        iota = jax.lax.broadcasted_iota(jnp.int32, x.shape, 1)
        idx = jnp.min(jnp.where(x == m, iota, _C), axis=1, keepdims=True)
        hit = (idx == t[:, j + 1:j + 2]).astype(jnp.float32)
        acc_row = acc_row + hit

    k = jax.lax.broadcasted_iota(jnp.int32, (1, 16), 1)
    kf = k.astype(jnp.float32)
    lows = jnp.where(k >= _N_BINS, 2.0, kf / _N_BINS)
    highs = jnp.where(k >= _N_BINS, 3.0, (kf + 1.0) / _N_BINS)
    in_bin = (conf > lows) & (conf <= highs)
    cnt = jnp.sum(in_bin.astype(jnp.float32), axis=0, keepdims=True)
    csum = jnp.sum(jnp.where(in_bin, conf, 0.0), axis=0, keepdims=True)
    asum = jnp.sum(jnp.where(in_bin, acc_row, 0.0), axis=0, keepdims=True)
    acc_ref[0:3, 0:16] += jnp.concatenate([cnt, csum, asum], axis=0)

    @pl.when(step == pl.num_programs(0) - 1)
    def _finish():
        a = acc_ref[0:3, 0:16]
        cnt_v = a[0:1, :]
        safe = jnp.maximum(cnt_v, 1.0)
        avg_conf = a[1:2, :] / safe
        avg_acc = a[2:3, :] / (safe * 3.0)
        term = jnp.abs(avg_conf - avg_acc) * (cnt_v / n_total)
        term = jnp.where(cnt_v > 0.0, term, 0.0)
        out_ref[...] = jnp.sum(term, axis=1, keepdims=True)


def kernel(logits, targets):
    n, h, c = logits.shape
    assert c == _C and h == 4 and n % _B == 0
    t32 = targets.astype(jnp.int32)
    import functools
    body = functools.partial(_ece_body, n_total=float(n))
    out = pl.pallas_call(
        body,
        grid=(n // _B,),
        in_specs=[
            pl.BlockSpec((_B, 4, _C), lambda i: (i, 0, 0)),
            pl.BlockSpec((_B, 4), lambda i: (i, 0)),
        ],
        out_specs=pl.BlockSpec((1, 1), lambda i: (0, 0)),
        out_shape=jax.ShapeDtypeStruct((1, 1), jnp.float32),
        scratch_shapes=[pltpu.VMEM((8, 128), jnp.float32)],
    )(logits, t32)
    return out.reshape(1)
